# baseline (speedup 1.0000x reference)
import jax
import jax.numpy as jnp
from jax.experimental import pallas as pl
from jax.experimental.pallas import tpu as pltpu

D_MODEL = 4096
NUM_E = 64
N_TOK = 4 * 4096
TOK_BLK = 1024
GRID = N_TOK // TOK_BLK


def _router_body(x_ref, wt_ref, b_ref, top1_ref, prob_ref, stats_ref):
    logits = jnp.dot(x_ref[...], wt_ref[...],
                     preferred_element_type=jnp.float32) + b_ref[...]
    m = jnp.max(logits, axis=-1, keepdims=True)
    e = jnp.exp(logits - m)
    s = jnp.sum(e, axis=-1, keepdims=True)
    rs = 1.0 / s
    top1 = jnp.argmax(logits, axis=-1).astype(jnp.int32)
    top1_ref[0, 0, :] = top1
    prob_ref[0, 0, :] = rs[:, 0]

    imp_part = jnp.sum(e * rs, axis=0)
    iota = jax.lax.broadcasted_iota(jnp.int32, (TOK_BLK, NUM_E), 1)
    cnt_part = jnp.sum((iota == top1[:, None]).astype(jnp.float32), axis=0)
    stats_ref[0, ...] = jnp.concatenate(
        [imp_part[None, :], cnt_part[None, :]], axis=0)


def _aux_body(stats_ref, aux_ref):
    st = stats_ref[...]
    imp = jnp.sum(st[:, 0, :], axis=0, keepdims=True)
    cnt = jnp.sum(st[:, 1, :], axis=0, keepdims=True)
    aux_ref[...] = (NUM_E / (N_TOK * N_TOK)) * jnp.sum(
        imp * cnt, axis=1, keepdims=True)


def kernel(x, W, b):
    xf = x.reshape(N_TOK, D_MODEL)
    wt = W.T
    b2 = b.reshape(1, NUM_E)
    top1, prob, stats = pl.pallas_call(
        _router_body,
        grid=(GRID,),
        in_specs=[
            pl.BlockSpec((TOK_BLK, D_MODEL), lambda i: (i, 0)),
            pl.BlockSpec((D_MODEL, NUM_E), lambda i: (0, 0)),
            pl.BlockSpec((1, NUM_E), lambda i: (0, 0)),
        ],
        out_specs=[
            pl.BlockSpec((1, 1, TOK_BLK), lambda i: (i, 0, 0)),
            pl.BlockSpec((1, 1, TOK_BLK), lambda i: (i, 0, 0)),
            pl.BlockSpec((1, 2, NUM_E), lambda i: (i, 0, 0)),
        ],
        out_shape=[
            jax.ShapeDtypeStruct((GRID, 1, TOK_BLK), jnp.int32),
            jax.ShapeDtypeStruct((GRID, 1, TOK_BLK), jnp.float32),
            jax.ShapeDtypeStruct((GRID, 2, NUM_E), jnp.float32),
        ],
        compiler_params=pltpu.CompilerParams(
            dimension_semantics=("parallel",),
        ),
    )(xf, wt, b2)
    aux = pl.pallas_call(
        _aux_body,
        out_shape=jax.ShapeDtypeStruct((1, 1), jnp.float32),
    )(stats)
    return (top1.reshape(x.shape[0], x.shape[1]),
            prob.reshape(x.shape[0], x.shape[1]),
            aux.reshape(()))


# speedup vs baseline: 1.1067x; 1.1067x over previous
import jax
import jax.numpy as jnp
from jax.experimental import pallas as pl
from jax.experimental.pallas import tpu as pltpu

D_MODEL = 4096
NUM_E = 64
N_TOK = 4 * 4096
TOK_BLK = 1024
GRID = N_TOK // TOK_BLK


def _router_body(x_ref, w_ref, b_ref, top1_ref, prob_ref, stats_ref):
    logits = jax.lax.dot_general(
        w_ref[...], x_ref[...], (((1,), (1,)), ((), ())),
        preferred_element_type=jnp.float32) + b_ref[...]
    m = jnp.max(logits, axis=0, keepdims=True)
    e = jnp.exp(logits - m)
    s = jnp.sum(e, axis=0, keepdims=True)
    rs = 1.0 / s
    top1 = jnp.argmax(logits, axis=0).astype(jnp.int32)
    top1_ref[0, 0, :] = top1
    prob_ref[0, 0, :] = rs[0, :]

    probs = e * rs
    imp_part = jnp.sum(probs, axis=1)
    iota = jax.lax.broadcasted_iota(jnp.int32, (NUM_E, TOK_BLK), 0)
    cnt_part = jnp.sum((iota == top1[None, :]).astype(jnp.float32), axis=1)
    stats_ref[0, ...] = jnp.concatenate(
        [imp_part[None, :], cnt_part[None, :]], axis=0)


def _aux_body(stats_ref, aux_ref):
    st = stats_ref[...]
    imp = jnp.sum(st[:, 0, :], axis=0, keepdims=True)
    cnt = jnp.sum(st[:, 1, :], axis=0, keepdims=True)
    aux_ref[...] = (NUM_E / (N_TOK * N_TOK)) * jnp.sum(
        imp * cnt, axis=1, keepdims=True)


def kernel(x, W, b):
    xf = x.reshape(N_TOK, D_MODEL)
    b2 = b.reshape(NUM_E, 1)
    top1, prob, stats = pl.pallas_call(
        _router_body,
        grid=(GRID,),
        in_specs=[
            pl.BlockSpec((TOK_BLK, D_MODEL), lambda i: (i, 0)),
            pl.BlockSpec((NUM_E, D_MODEL), lambda i: (0, 0)),
            pl.BlockSpec((NUM_E, 1), lambda i: (0, 0)),
        ],
        out_specs=[
            pl.BlockSpec((1, 1, TOK_BLK), lambda i: (i, 0, 0)),
            pl.BlockSpec((1, 1, TOK_BLK), lambda i: (i, 0, 0)),
            pl.BlockSpec((1, 2, NUM_E), lambda i: (i, 0, 0)),
        ],
        out_shape=[
            jax.ShapeDtypeStruct((GRID, 1, TOK_BLK), jnp.int32),
            jax.ShapeDtypeStruct((GRID, 1, TOK_BLK), jnp.float32),
            jax.ShapeDtypeStruct((GRID, 2, NUM_E), jnp.float32),
        ],
        compiler_params=pltpu.CompilerParams(
            dimension_semantics=("parallel",),
        ),
    )(xf, W, b2)
    aux = pl.pallas_call(
        _aux_body,
        out_shape=jax.ShapeDtypeStruct((1, 1), jnp.float32),
    )(stats)
    return (top1.reshape(x.shape[0], x.shape[1]),
            prob.reshape(x.shape[0], x.shape[1]),
            aux.reshape(()))


# speedup vs baseline: 1.1117x; 1.0046x over previous
import jax
import jax.numpy as jnp
from jax.experimental import pallas as pl
from jax.experimental.pallas import tpu as pltpu

D_MODEL = 4096
NUM_E = 64
N_TOK = 4 * 4096
TOK_BLK = 1024
HALF = TOK_BLK // 2
GRID = N_TOK // TOK_BLK


def _half(x_half, w, b, top1_ref, prob_ref, lo):
    logits = jax.lax.dot_general(
        w, x_half, (((1,), (1,)), ((), ())),
        preferred_element_type=jnp.float32) + b
    m = jnp.max(logits, axis=0, keepdims=True)
    e = jnp.exp(logits - m)
    s = jnp.sum(e, axis=0, keepdims=True)
    rs = 1.0 / s
    top1 = jnp.argmax(logits, axis=0).astype(jnp.int32)
    top1_ref[0, 0, pl.ds(lo, HALF)] = top1
    prob_ref[0, 0, pl.ds(lo, HALF)] = rs[0, :]

    probs = e * rs
    imp_part = jnp.sum(probs, axis=1)
    iota = jax.lax.broadcasted_iota(jnp.int32, (NUM_E, HALF), 0)
    cnt_part = jnp.sum((iota == top1[None, :]).astype(jnp.float32), axis=1)
    return imp_part, cnt_part


def _router_body(xa_ref, xb_ref, w_ref, b_ref, top1_ref, prob_ref, stats_ref):
    w = w_ref[...]
    b = b_ref[...]
    imp0, cnt0 = _half(xa_ref[...], w, b, top1_ref, prob_ref, 0)
    imp1, cnt1 = _half(xb_ref[...], w, b, top1_ref, prob_ref, HALF)
    stats_ref[0, ...] = jnp.concatenate(
        [(imp0 + imp1)[None, :], (cnt0 + cnt1)[None, :]], axis=0)


def _aux_body(stats_ref, aux_ref):
    st = stats_ref[...]
    imp = jnp.sum(st[:, 0, :], axis=0, keepdims=True)
    cnt = jnp.sum(st[:, 1, :], axis=0, keepdims=True)
    aux_ref[...] = (NUM_E / (N_TOK * N_TOK)) * jnp.sum(
        imp * cnt, axis=1, keepdims=True)


def kernel(x, W, b):
    xf = x.reshape(N_TOK, D_MODEL)
    b2 = b.reshape(NUM_E, 1)
    top1, prob, stats = pl.pallas_call(
        _router_body,
        grid=(GRID,),
        in_specs=[
            pl.BlockSpec((HALF, D_MODEL), lambda i: (2 * i, 0)),
            pl.BlockSpec((HALF, D_MODEL), lambda i: (2 * i + 1, 0)),
            pl.BlockSpec((NUM_E, D_MODEL), lambda i: (0, 0)),
            pl.BlockSpec((NUM_E, 1), lambda i: (0, 0)),
        ],
        out_specs=[
            pl.BlockSpec((1, 1, TOK_BLK), lambda i: (i, 0, 0)),
            pl.BlockSpec((1, 1, TOK_BLK), lambda i: (i, 0, 0)),
            pl.BlockSpec((1, 2, NUM_E), lambda i: (i, 0, 0)),
        ],
        out_shape=[
            jax.ShapeDtypeStruct((GRID, 1, TOK_BLK), jnp.int32),
            jax.ShapeDtypeStruct((GRID, 1, TOK_BLK), jnp.float32),
            jax.ShapeDtypeStruct((GRID, 2, NUM_E), jnp.float32),
        ],
        compiler_params=pltpu.CompilerParams(
            dimension_semantics=("arbitrary",),
        ),
    )(xf, xf, W, b2)
    aux = pl.pallas_call(
        _aux_body,
        out_shape=jax.ShapeDtypeStruct((1, 1), jnp.float32),
    )(stats)
    return (top1.reshape(x.shape[0], x.shape[1]),
            prob.reshape(x.shape[0], x.shape[1]),
            aux.reshape(()))


# speedup vs baseline: 1.1285x; 1.0151x over previous
import jax
import jax.numpy as jnp
from jax.experimental import pallas as pl
from jax.experimental.pallas import tpu as pltpu

D_MODEL = 4096
NUM_E = 64
N_TOK = 4 * 4096
TOK_BLK = 1024
GRID = N_TOK // TOK_BLK


def _router_body(x_ref, w_ref, b_ref, o_ref, aux_ref, acc_ref):
    i = pl.program_id(0)
    logits = jax.lax.dot_general(
        w_ref[...], x_ref[...], (((1,), (1,)), ((), ())),
        preferred_element_type=jnp.float32) + b_ref[...]
    m = jnp.max(logits, axis=0, keepdims=True)
    e = jnp.exp(logits - m)
    s = jnp.sum(e, axis=0, keepdims=True)
    rs = 1.0 / s
    top1 = jnp.argmax(logits, axis=0).astype(jnp.int32)
    o_ref[0, 0, :] = top1
    o_ref[0, 1, :] = jax.lax.bitcast_convert_type(rs[0, :], jnp.int32)

    probs = e * rs
    imp_part = jnp.sum(probs, axis=1)
    iota = jax.lax.broadcasted_iota(jnp.int32, (NUM_E, TOK_BLK), 0)
    cnt_part = jnp.sum((iota == top1[None, :]).astype(jnp.float32), axis=1)
    part = jnp.concatenate([imp_part[None, :], cnt_part[None, :]], axis=0)

    @pl.when(i == 0)
    def _init():
        acc_ref[...] = part

    @pl.when(i > 0)
    def _accum():
        acc_ref[...] += part

    @pl.when(i == GRID - 1)
    def _finish():
        st = acc_ref[...]
        aux_ref[...] = (NUM_E / (N_TOK * N_TOK)) * jnp.sum(
            st[0:1, :] * st[1:2, :], axis=1, keepdims=True)


def kernel(x, W, b):
    xf = x.reshape(N_TOK, D_MODEL)
    b2 = b.reshape(NUM_E, 1)
    out, aux = pl.pallas_call(
        _router_body,
        grid=(GRID,),
        in_specs=[
            pl.BlockSpec((TOK_BLK, D_MODEL), lambda i: (i, 0)),
            pl.BlockSpec((NUM_E, D_MODEL), lambda i: (0, 0)),
            pl.BlockSpec((NUM_E, 1), lambda i: (0, 0)),
        ],
        out_specs=[
            pl.BlockSpec((1, 2, TOK_BLK), lambda i: (i, 0, 0)),
            pl.BlockSpec((1, 1), lambda i: (0, 0)),
        ],
        out_shape=[
            jax.ShapeDtypeStruct((GRID, 2, TOK_BLK), jnp.int32),
            jax.ShapeDtypeStruct((1, 1), jnp.float32),
        ],
        scratch_shapes=[pltpu.VMEM((2, NUM_E), jnp.float32)],
        compiler_params=pltpu.CompilerParams(
            dimension_semantics=("arbitrary",),
        ),
    )(xf, W, b2)
    top1 = out[:, 0, :].reshape(x.shape[0], x.shape[1])
    prob = jax.lax.bitcast_convert_type(
        out[:, 1, :], jnp.float32).reshape(x.shape[0], x.shape[1])
    return (top1, prob, aux.reshape(()))
